# Initial kernel scaffold; baseline (speedup 1.0000x reference)
import jax
import jax.numpy as jnp
from jax.experimental import pallas as pl


def kernel(x, edge_index, edge_weight, batch, W, b):
    raise NotImplementedError("write your pallas kernel here")


# speedup vs baseline: 55.6045x; 55.6045x over previous
import functools
import jax
import jax.numpy as jnp
from jax import lax
from jax.experimental import pallas as pl
from jax.experimental.pallas import tpu as pltpu
from jax.experimental.pallas import tpu_sc as plsc

N_ = 10000
E_ = 320000
D_ = 128
K_ = 64
B_ = 8
BK_ = B_ * K_
DA_ = 128
NCH_ = 10
NC_BLK = N_ // NCH_
ECH_ = 160
EC_BLK = E_ // ECH_
SC_CHUNK = 400


def _s_body(x_ref, w_ref, b_ref, bat_ref, out_ref):
    xb = x_ref[...]
    logits = jnp.dot(xb, w_ref[...], preferred_element_type=jnp.float32)
    logits = logits + b_ref[...]
    m = jnp.max(logits, axis=1, keepdims=True)
    e = jnp.exp(logits - m)
    s1 = e / jnp.sum(e, axis=1, keepdims=True)
    m2 = jnp.max(s1, axis=1, keepdims=True)
    e2 = jnp.exp(s1 - m2)
    s2 = e2 / jnp.sum(e2, axis=1, keepdims=True)
    pad = jnp.zeros((NC_BLK, DA_ - K_ - 1), dtype=jnp.float32)
    out_ref[...] = jnp.concatenate([s2, bat_ref[...], pad], axis=1)


def _s_kernel(x, W, b2, batch_f):
    return pl.pallas_call(
        _s_body,
        grid=(NCH_,),
        in_specs=[
            pl.BlockSpec((NC_BLK, D_), lambda i: (i, 0)),
            pl.BlockSpec((D_, K_), lambda i: (0, 0)),
            pl.BlockSpec((1, K_), lambda i: (0, 0)),
            pl.BlockSpec((NC_BLK, 1), lambda i: (i, 0)),
        ],
        out_specs=pl.BlockSpec((NC_BLK, DA_), lambda i: (i, 0)),
        out_shape=jax.ShapeDtypeStruct((N_, DA_), jnp.float32),
    )(x, W, b2, batch_f)


def _gather_body(table_hbm, src_hbm, dst_hbm, gs_hbm, gd_hbm,
                 idx_v, rows_v, sem):
    info = plsc.get_sparse_core_info()
    nc = info.num_cores
    wid = lax.axis_index("s") * nc + lax.axis_index("c")
    per_tile = E_ // (nc * info.num_subcores)
    base = wid * per_tile
    nchunks = per_tile // SC_CHUNK
    for idx_hbm, out_hbm in ((src_hbm, gs_hbm), (dst_hbm, gd_hbm)):
        for j in range(nchunks):
            off = base + j * SC_CHUNK
            pltpu.sync_copy(idx_hbm.at[pl.ds(off, SC_CHUNK)], idx_v)
            pltpu.async_copy(table_hbm.at[idx_v], rows_v, sem).wait()
            pltpu.sync_copy(rows_v, out_hbm.at[pl.ds(off, SC_CHUNK)])


def _gather_kernel(s_aug, src, dst):
    mesh = plsc.VectorSubcoreMesh(core_axis_name="c", subcore_axis_name="s")
    fn = functools.partial(
        pl.kernel,
        mesh=mesh,
        out_type=[
            jax.ShapeDtypeStruct((E_, DA_), jnp.float32),
            jax.ShapeDtypeStruct((E_, DA_), jnp.float32),
        ],
        scratch_types=[
            pltpu.VMEM((SC_CHUNK,), jnp.int32),
            pltpu.VMEM((SC_CHUNK, DA_), jnp.float32),
            pltpu.SemaphoreType.DMA,
        ],
    )(_gather_body)
    return fn(s_aug, src, dst)


def _edge_body(gs_ref, gd_ref, w_ref, adj_ref, scal_ref, den_ref):
    i = pl.program_id(0)

    @pl.when(i == 0)
    def _init():
        adj_ref[...] = jnp.zeros((BK_, BK_), jnp.float32)
        den_ref[0, 0] = 0.0

    gs = gs_ref[...]
    gd = gd_ref[...]
    w = w_ref[...]
    ss = gs[:, :K_]
    sd = gd[:, :K_]
    bs = gs[:, K_:K_ + 1]
    bd = gd[:, K_:K_ + 1]
    colb = (lax.broadcasted_iota(jnp.int32, (1, BK_), 1) // K_
            ).astype(jnp.float32)
    tiled_s = jnp.concatenate([ss] * B_, axis=1)
    tiled_d = jnp.concatenate([sd] * B_, axis=1)
    es = jnp.where(colb == bs, tiled_s, 0.0)
    ed = jnp.where(colb == bd, tiled_d * w, 0.0)
    adj_ref[...] += lax.dot_general(
        es, ed, (((0,), (0,)), ((), ())),
        preferred_element_type=jnp.float32)
    den_ref[0, 0] += jnp.sum(w * jnp.sum(sd * sd, axis=1, keepdims=True))

    @pl.when(i == ECH_ - 1)
    def _fin():
        adj = adj_ref[...]
        r = lax.broadcasted_iota(jnp.int32, (BK_, BK_), 0)
        c = lax.broadcasted_iota(jnp.int32, (BK_, BK_), 1)
        trace = jnp.sum(jnp.where(r == c, adj, 0.0))
        r8 = lax.broadcasted_iota(jnp.int32, (8, 128), 0)
        c8 = lax.broadcasted_iota(jnp.int32, (8, 128), 1)
        scal_ref[...] = (jnp.where((r8 == 0) & (c8 == 0), trace, 0.0)
                         + jnp.where((r8 == 0) & (c8 == 1), den_ref[0, 0],
                                     0.0))


def _edge_kernel(gs, gd, w2):
    return pl.pallas_call(
        _edge_body,
        grid=(ECH_,),
        in_specs=[
            pl.BlockSpec((EC_BLK, DA_), lambda i: (i, 0)),
            pl.BlockSpec((EC_BLK, DA_), lambda i: (i, 0)),
            pl.BlockSpec((EC_BLK, 1), lambda i: (i, 0)),
        ],
        out_specs=[
            pl.BlockSpec((BK_, BK_), lambda i: (0, 0)),
            pl.BlockSpec((8, 128), lambda i: (0, 0)),
        ],
        out_shape=[
            jax.ShapeDtypeStruct((BK_, BK_), jnp.float32),
            jax.ShapeDtypeStruct((8, 128), jnp.float32),
        ],
        scratch_shapes=[pltpu.SMEM((1, 1), jnp.float32)],
    )(gs, gd, w2)


_SELU_ALPHA = 1.6732632423543772
_SELU_SCALE = 1.0507009873554805


def _node_body(saug_ref, x_ref, bat_ref, adj_ref,
               outx_ref, adjn_ref, ortho_ref, cc_acc, xc_acc):
    i = pl.program_id(0)

    @pl.when(i == 0)
    def _init():
        cc_acc[...] = jnp.zeros((BK_, BK_), jnp.float32)
        xc_acc[...] = jnp.zeros((BK_, D_), jnp.float32)

    s = saug_ref[:, :K_]
    bat = bat_ref[...]
    colb = (lax.broadcasted_iota(jnp.int32, (1, BK_), 1) // K_
            ).astype(jnp.float32)
    cfull = jnp.where(colb == bat, jnp.concatenate([s] * B_, axis=1), 0.0)
    cc_acc[...] += lax.dot_general(
        cfull, cfull, (((0,), (0,)), ((), ())),
        preferred_element_type=jnp.float32)
    xc_acc[...] += lax.dot_general(
        cfull, x_ref[...], (((0,), (0,)), ((), ())),
        preferred_element_type=jnp.float32)

    @pl.when(i == NCH_ - 1)
    def _fin():
        r = lax.broadcasted_iota(jnp.int32, (BK_, BK_), 0)
        c = lax.broadcasted_iota(jnp.int32, (BK_, BK_), 1)
        eye = jnp.where(r == c, 1.0, 0.0).astype(jnp.float32)
        adj = adj_ref[...] * (1.0 - eye)
        srow = jnp.sum(adj, axis=1, keepdims=True)
        inv = 1.0 / (jnp.sqrt(srow) + 1e-12)
        inv_row = jnp.sum(eye * inv, axis=0, keepdims=True)
        adjn_ref[...] = adj * inv * inv_row
        cc = cc_acc[...]
        seg = jnp.where(
            lax.broadcasted_iota(jnp.int32, (B_, BK_), 0)
            == lax.broadcasted_iota(jnp.int32, (B_, BK_), 1) // K_,
            1.0, 0.0).astype(jnp.float32)
        rowsq = jnp.sum(cc * cc, axis=1, keepdims=True)
        norm = jnp.sqrt(lax.dot_general(
            seg, rowsq, (((1,), (0,)), ((), ())),
            preferred_element_type=jnp.float32))
        col_scale = jnp.sum(seg * (1.0 / norm), axis=0,
                            keepdims=True)
        ccs = cc * col_scale
        dmat = ccs - eye / jnp.sqrt(jnp.float32(K_))
        rowsd = jnp.sum(dmat * dmat, axis=1, keepdims=True)
        segd = lax.dot_general(seg, rowsd, (((1,), (0,)), ((), ())),
                               preferred_element_type=jnp.float32)
        ortho = jnp.mean(jnp.sqrt(segd))
        r8 = lax.broadcasted_iota(jnp.int32, (8, 128), 0)
        c8 = lax.broadcasted_iota(jnp.int32, (8, 128), 1)
        ortho_ref[...] = jnp.where((r8 == 0) & (c8 == 0), ortho, 0.0)
        xc = xc_acc[...]
        outx_ref[...] = _SELU_SCALE * jnp.where(
            xc > 0.0, xc, _SELU_ALPHA * (jnp.exp(xc) - 1.0))


def _node_kernel(s_aug, x, batch_f, adj_raw):
    return pl.pallas_call(
        _node_body,
        grid=(NCH_,),
        in_specs=[
            pl.BlockSpec((NC_BLK, DA_), lambda i: (i, 0)),
            pl.BlockSpec((NC_BLK, D_), lambda i: (i, 0)),
            pl.BlockSpec((NC_BLK, 1), lambda i: (i, 0)),
            pl.BlockSpec((BK_, BK_), lambda i: (0, 0)),
        ],
        out_specs=[
            pl.BlockSpec((BK_, D_), lambda i: (0, 0)),
            pl.BlockSpec((BK_, BK_), lambda i: (0, 0)),
            pl.BlockSpec((8, 128), lambda i: (0, 0)),
        ],
        out_shape=[
            jax.ShapeDtypeStruct((BK_, D_), jnp.float32),
            jax.ShapeDtypeStruct((BK_, BK_), jnp.float32),
            jax.ShapeDtypeStruct((8, 128), jnp.float32),
        ],
        scratch_shapes=[
            pltpu.VMEM((BK_, BK_), jnp.float32),
            pltpu.VMEM((BK_, D_), jnp.float32),
        ],
    )(s_aug, x, batch_f, adj_raw)


def kernel(x, edge_index, edge_weight, batch, W, b):
    src = edge_index[0]
    dst = edge_index[1]
    batch_f = batch.astype(jnp.float32).reshape(N_, 1)
    b2 = b.reshape(1, K_)
    s_aug = _s_kernel(x, W, b2, batch_f)
    gs, gd = _gather_kernel(s_aug, src, dst)
    w2 = edge_weight.reshape(E_, 1)
    adj_raw, scal = _edge_kernel(gs, gd, w2)
    mincut_loss = -scal[0, 0] / scal[0, 1]
    out_x, adj_norm, ortho_scal = _node_kernel(s_aug, x, batch_f, adj_raw)
    return out_x, adj_norm, mincut_loss, ortho_scal[0, 0]


# speedup vs baseline: 57.7684x; 1.0389x over previous
import functools
import jax
import jax.numpy as jnp
from jax import lax
from jax.experimental import pallas as pl
from jax.experimental.pallas import tpu as pltpu
from jax.experimental.pallas import tpu_sc as plsc

N_ = 10000
E_ = 320000
D_ = 128
K_ = 64
B_ = 8
BK_ = B_ * K_
DA_ = 128
NCH_ = 10
NC_BLK = N_ // NCH_
ECH_ = 160
EC_BLK = E_ // ECH_
SC_CHUNK = 400


def _s_body(x_ref, w_ref, b_ref, bat_ref, out_ref):
    xb = x_ref[...]
    logits = jnp.dot(xb, w_ref[...], preferred_element_type=jnp.float32)
    logits = logits + b_ref[...]
    m = jnp.max(logits, axis=1, keepdims=True)
    e = jnp.exp(logits - m)
    s1 = e / jnp.sum(e, axis=1, keepdims=True)
    m2 = jnp.max(s1, axis=1, keepdims=True)
    e2 = jnp.exp(s1 - m2)
    s2 = e2 / jnp.sum(e2, axis=1, keepdims=True)
    pad = jnp.zeros((NC_BLK, DA_ - K_ - 1), dtype=jnp.float32)
    out_ref[...] = jnp.concatenate([s2, bat_ref[...], pad], axis=1)


def _s_kernel(x, W, b2, batch_f):
    return pl.pallas_call(
        _s_body,
        grid=(NCH_,),
        in_specs=[
            pl.BlockSpec((NC_BLK, D_), lambda i: (i, 0)),
            pl.BlockSpec((D_, K_), lambda i: (0, 0)),
            pl.BlockSpec((1, K_), lambda i: (0, 0)),
            pl.BlockSpec((NC_BLK, 1), lambda i: (i, 0)),
        ],
        out_specs=pl.BlockSpec((NC_BLK, DA_), lambda i: (i, 0)),
        out_shape=jax.ShapeDtypeStruct((N_, DA_), jnp.float32),
    )(x, W, b2, batch_f)


def _gather_body(table_hbm, src_hbm, dst_hbm, gs_hbm, gd_hbm,
                 idx_v, rows_v, sem):
    info = plsc.get_sparse_core_info()
    nc = info.num_cores
    wid = lax.axis_index("s") * nc + lax.axis_index("c")
    per_tile = E_ // (nc * info.num_subcores)
    base = wid * per_tile
    nchunks = per_tile // SC_CHUNK
    for idx_hbm, out_hbm in ((src_hbm, gs_hbm), (dst_hbm, gd_hbm)):
        for j in range(nchunks):
            off = base + j * SC_CHUNK
            pltpu.sync_copy(idx_hbm.at[pl.ds(off, SC_CHUNK)], idx_v)
            pltpu.async_copy(table_hbm.at[idx_v], rows_v, sem).wait()
            pltpu.sync_copy(rows_v, out_hbm.at[pl.ds(off, SC_CHUNK)])


def _gather_kernel(s_aug, src, dst):
    mesh = plsc.VectorSubcoreMesh(core_axis_name="c", subcore_axis_name="s")
    fn = functools.partial(
        pl.kernel,
        mesh=mesh,
        out_type=[
            jax.ShapeDtypeStruct((E_, DA_), jnp.float32),
            jax.ShapeDtypeStruct((E_, DA_), jnp.float32),
        ],
        scratch_types=[
            pltpu.VMEM((SC_CHUNK,), jnp.int32),
            pltpu.VMEM((SC_CHUNK, DA_), jnp.float32),
            pltpu.SemaphoreType.DMA,
        ],
    )(_gather_body)
    return fn(s_aug, src, dst)


def _edge_body(gs_ref, gd_ref, w_ref, adjn_ref, scal_ref, adj_ref, den_ref):
    i = pl.program_id(0)

    @pl.when(i == 0)
    def _init():
        adj_ref[...] = jnp.zeros((BK_, BK_), jnp.float32)
        den_ref[0, 0] = 0.0

    gs = gs_ref[...]
    gd = gd_ref[...]
    w = w_ref[...]
    ss = gs[:, :K_]
    sd = gd[:, :K_]
    bs = gs[:, K_:K_ + 1]
    bd = gd[:, K_:K_ + 1]
    colb = (lax.broadcasted_iota(jnp.int32, (1, BK_), 1) // K_
            ).astype(jnp.float32)
    tiled_s = jnp.concatenate([ss] * B_, axis=1)
    tiled_d = jnp.concatenate([sd] * B_, axis=1)
    es = jnp.where(colb == bs, tiled_s, 0.0).astype(jnp.bfloat16)
    ed = jnp.where(colb == bd, tiled_d * w, 0.0).astype(jnp.bfloat16)
    adj_ref[...] += lax.dot_general(
        es, ed, (((0,), (0,)), ((), ())),
        preferred_element_type=jnp.float32)
    den_ref[0, 0] += jnp.sum(w * jnp.sum(sd * sd, axis=1, keepdims=True))

    @pl.when(i == ECH_ - 1)
    def _fin():
        r = lax.broadcasted_iota(jnp.int32, (BK_, BK_), 0)
        c = lax.broadcasted_iota(jnp.int32, (BK_, BK_), 1)
        eye = jnp.where(r == c, 1.0, 0.0).astype(jnp.float32)
        adj_raw = adj_ref[...]
        trace = jnp.sum(adj_raw * eye)
        adj = adj_raw * (1.0 - eye)
        srow = jnp.sum(adj, axis=1, keepdims=True)
        inv = 1.0 / (jnp.sqrt(srow) + 1e-12)
        inv_row = jnp.sum(eye * inv, axis=0, keepdims=True)
        adjn_ref[...] = adj * inv * inv_row
        r8 = lax.broadcasted_iota(jnp.int32, (8, 128), 0)
        c8 = lax.broadcasted_iota(jnp.int32, (8, 128), 1)
        scal_ref[...] = (jnp.where((r8 == 0) & (c8 == 0), trace, 0.0)
                         + jnp.where((r8 == 0) & (c8 == 1), den_ref[0, 0],
                                     0.0))


def _edge_kernel(gs, gd, w2):
    return pl.pallas_call(
        _edge_body,
        grid=(ECH_,),
        in_specs=[
            pl.BlockSpec((EC_BLK, DA_), lambda i: (i, 0)),
            pl.BlockSpec((EC_BLK, DA_), lambda i: (i, 0)),
            pl.BlockSpec((EC_BLK, 1), lambda i: (i, 0)),
        ],
        out_specs=[
            pl.BlockSpec((BK_, BK_), lambda i: (0, 0)),
            pl.BlockSpec((8, 128), lambda i: (0, 0)),
        ],
        out_shape=[
            jax.ShapeDtypeStruct((BK_, BK_), jnp.float32),
            jax.ShapeDtypeStruct((8, 128), jnp.float32),
        ],
        scratch_shapes=[
            pltpu.VMEM((BK_, BK_), jnp.float32),
            pltpu.SMEM((1, 1), jnp.float32),
        ],
    )(gs, gd, w2)


_SELU_ALPHA = 1.6732632423543772
_SELU_SCALE = 1.0507009873554805


def _node_body(saug_ref, x_ref, bat_ref,
               outx_ref, ortho_ref, cc_acc, xc_acc):
    i = pl.program_id(0)

    @pl.when(i == 0)
    def _init():
        cc_acc[...] = jnp.zeros((BK_, BK_), jnp.float32)
        xc_acc[...] = jnp.zeros((BK_, D_), jnp.float32)

    s = saug_ref[:, :K_]
    bat = bat_ref[...]
    colb = (lax.broadcasted_iota(jnp.int32, (1, BK_), 1) // K_
            ).astype(jnp.float32)
    cfull = jnp.where(colb == bat, jnp.concatenate([s] * B_, axis=1), 0.0)
    cc_acc[...] += lax.dot_general(
        cfull, cfull, (((0,), (0,)), ((), ())),
        preferred_element_type=jnp.float32)
    xc_acc[...] += lax.dot_general(
        cfull, x_ref[...], (((0,), (0,)), ((), ())),
        preferred_element_type=jnp.float32)

    @pl.when(i == NCH_ - 1)
    def _fin():
        r = lax.broadcasted_iota(jnp.int32, (BK_, BK_), 0)
        c = lax.broadcasted_iota(jnp.int32, (BK_, BK_), 1)
        eye = jnp.where(r == c, 1.0, 0.0).astype(jnp.float32)
        cc = cc_acc[...]
        seg = jnp.where(
            lax.broadcasted_iota(jnp.int32, (B_, BK_), 0)
            == lax.broadcasted_iota(jnp.int32, (B_, BK_), 1) // K_,
            1.0, 0.0).astype(jnp.float32)
        rowsq = jnp.sum(cc * cc, axis=1, keepdims=True)
        norm = jnp.sqrt(lax.dot_general(
            seg, rowsq, (((1,), (0,)), ((), ())),
            preferred_element_type=jnp.float32))
        col_scale = jnp.sum(seg * (1.0 / norm), axis=0,
                            keepdims=True)
        ccs = cc * col_scale
        dmat = ccs - eye / jnp.sqrt(jnp.float32(K_))
        rowsd = jnp.sum(dmat * dmat, axis=1, keepdims=True)
        segd = lax.dot_general(seg, rowsd, (((1,), (0,)), ((), ())),
                               preferred_element_type=jnp.float32)
        ortho = jnp.mean(jnp.sqrt(segd))
        r8 = lax.broadcasted_iota(jnp.int32, (8, 128), 0)
        c8 = lax.broadcasted_iota(jnp.int32, (8, 128), 1)
        ortho_ref[...] = jnp.where((r8 == 0) & (c8 == 0), ortho, 0.0)
        xc = xc_acc[...]
        outx_ref[...] = _SELU_SCALE * jnp.where(
            xc > 0.0, xc, _SELU_ALPHA * (jnp.exp(xc) - 1.0))


def _node_kernel(s_aug, x, batch_f):
    return pl.pallas_call(
        _node_body,
        grid=(NCH_,),
        in_specs=[
            pl.BlockSpec((NC_BLK, DA_), lambda i: (i, 0)),
            pl.BlockSpec((NC_BLK, D_), lambda i: (i, 0)),
            pl.BlockSpec((NC_BLK, 1), lambda i: (i, 0)),
        ],
        out_specs=[
            pl.BlockSpec((BK_, D_), lambda i: (0, 0)),
            pl.BlockSpec((8, 128), lambda i: (0, 0)),
        ],
        out_shape=[
            jax.ShapeDtypeStruct((BK_, D_), jnp.float32),
            jax.ShapeDtypeStruct((8, 128), jnp.float32),
        ],
        scratch_shapes=[
            pltpu.VMEM((BK_, BK_), jnp.float32),
            pltpu.VMEM((BK_, D_), jnp.float32),
        ],
    )(s_aug, x, batch_f)


def kernel(x, edge_index, edge_weight, batch, W, b):
    src = edge_index[0]
    dst = edge_index[1]
    batch_f = batch.astype(jnp.float32).reshape(N_, 1)
    b2 = b.reshape(1, K_)
    s_aug = _s_kernel(x, W, b2, batch_f)
    gs, gd = _gather_kernel(s_aug, src, dst)
    w2 = edge_weight.reshape(E_, 1)
    adj_norm, scal = _edge_kernel(gs, gd, w2)
    mincut_loss = -scal[0, 0] / scal[0, 1]
    out_x, ortho_scal = _node_kernel(s_aug, x, batch_f)
    return out_x, adj_norm, mincut_loss, ortho_scal[0, 0]


# speedup vs baseline: 60.3102x; 1.0440x over previous
import functools
import jax
import jax.numpy as jnp
from jax import lax
from jax.experimental import pallas as pl
from jax.experimental.pallas import tpu as pltpu
from jax.experimental.pallas import tpu_sc as plsc

N_ = 10000
E_ = 320000
D_ = 128
K_ = 64
B_ = 8
BK_ = B_ * K_
DA_ = 128
NCH_ = 10
NC_BLK = N_ // NCH_
ECH_ = 160
EC_BLK = E_ // ECH_
SC_CHUNK = 400


def _s_body(x_ref, w_ref, b_ref, bat_ref, out_ref):
    xb = x_ref[...]
    logits = jnp.dot(xb, w_ref[...], preferred_element_type=jnp.float32)
    logits = logits + b_ref[...]
    m = jnp.max(logits, axis=1, keepdims=True)
    e = jnp.exp(logits - m)
    s1 = e / jnp.sum(e, axis=1, keepdims=True)
    m2 = jnp.max(s1, axis=1, keepdims=True)
    e2 = jnp.exp(s1 - m2)
    s2 = e2 / jnp.sum(e2, axis=1, keepdims=True)
    pad = jnp.zeros((NC_BLK, DA_ - K_ - 1), dtype=jnp.float32)
    out_ref[...] = jnp.concatenate([s2, bat_ref[...], pad], axis=1)


def _s_kernel(x, W, b2, batch_f):
    return pl.pallas_call(
        _s_body,
        grid=(NCH_,),
        in_specs=[
            pl.BlockSpec((NC_BLK, D_), lambda i: (i, 0)),
            pl.BlockSpec((D_, K_), lambda i: (0, 0)),
            pl.BlockSpec((1, K_), lambda i: (0, 0)),
            pl.BlockSpec((NC_BLK, 1), lambda i: (i, 0)),
        ],
        out_specs=pl.BlockSpec((NC_BLK, DA_), lambda i: (i, 0)),
        out_shape=jax.ShapeDtypeStruct((N_, DA_), jnp.float32),
    )(x, W, b2, batch_f)


def _gather_body(table_hbm, src_hbm, dst_hbm, gs_hbm, gd_hbm,
                 idx0, idx1, rows0, rows1, gsem0, gsem1, osem0, osem1):
    info = plsc.get_sparse_core_info()
    nc = info.num_cores
    wid = lax.axis_index("s") * nc + lax.axis_index("c")
    per_tile = E_ // (nc * info.num_subcores)
    base = wid * per_tile
    nchunks = per_tile // SC_CHUNK
    idxb = (idx0, idx1)
    rowsb = (rows0, rows1)
    gsem = (gsem0, gsem1)
    osem = (osem0, osem1)
    plan = [(src_hbm, gs_hbm, j) for j in range(nchunks)] + \
           [(dst_hbm, gd_hbm, j) for j in range(nchunks)]
    g = [None, None]
    o = [None, None]
    for j, (idx_hbm, out_hbm, c) in enumerate(plan):
        bi = j & 1
        if j >= 2:
            o[bi].wait()
        off = base + c * SC_CHUNK
        pltpu.sync_copy(idx_hbm.at[pl.ds(off, SC_CHUNK)], idxb[bi])
        g[bi] = pltpu.async_copy(table_hbm.at[idxb[bi]], rowsb[bi], gsem[bi])
        if j >= 1:
            _, pout, pc = plan[j - 1]
            pbi = 1 - bi
            g[pbi].wait()
            o[pbi] = pltpu.async_copy(
                rowsb[pbi], pout.at[pl.ds(base + pc * SC_CHUNK, SC_CHUNK)],
                osem[pbi])
    lbi = (len(plan) - 1) & 1
    _, lout, lc = plan[-1]
    g[lbi].wait()
    o[lbi] = pltpu.async_copy(
        rowsb[lbi], lout.at[pl.ds(base + lc * SC_CHUNK, SC_CHUNK)],
        osem[lbi])
    o[0].wait()
    o[1].wait()


def _gather_kernel(s_aug, src, dst):
    mesh = plsc.VectorSubcoreMesh(core_axis_name="c", subcore_axis_name="s")
    fn = functools.partial(
        pl.kernel,
        mesh=mesh,
        out_type=[
            jax.ShapeDtypeStruct((E_, DA_), jnp.float32),
            jax.ShapeDtypeStruct((E_, DA_), jnp.float32),
        ],
        scratch_types=[
            pltpu.VMEM((SC_CHUNK,), jnp.int32),
            pltpu.VMEM((SC_CHUNK,), jnp.int32),
            pltpu.VMEM((SC_CHUNK, DA_), jnp.float32),
            pltpu.VMEM((SC_CHUNK, DA_), jnp.float32),
            pltpu.SemaphoreType.DMA,
            pltpu.SemaphoreType.DMA,
            pltpu.SemaphoreType.DMA,
            pltpu.SemaphoreType.DMA,
        ],
    )(_gather_body)
    return fn(s_aug, src, dst)


def _edge_body(gs_ref, gd_ref, w_ref, adjn_ref, scal_ref, adj_ref, den_ref):
    i = pl.program_id(0)

    @pl.when(i == 0)
    def _init():
        adj_ref[...] = jnp.zeros((BK_, BK_), jnp.float32)
        den_ref[0, 0] = 0.0

    gs = gs_ref[...]
    gd = gd_ref[...]
    w = w_ref[...]
    ss = gs[:, :K_]
    sd = gd[:, :K_]
    bs = gs[:, K_:K_ + 1]
    bd = gd[:, K_:K_ + 1]
    colb = (lax.broadcasted_iota(jnp.int32, (1, BK_), 1) // K_
            ).astype(jnp.float32)
    tiled_s = jnp.concatenate([ss] * B_, axis=1)
    tiled_d = jnp.concatenate([sd] * B_, axis=1)
    es = jnp.where(colb == bs, tiled_s, 0.0).astype(jnp.bfloat16)
    ed = jnp.where(colb == bd, tiled_d * w, 0.0).astype(jnp.bfloat16)
    adj_ref[...] += lax.dot_general(
        es, ed, (((0,), (0,)), ((), ())),
        preferred_element_type=jnp.float32)
    den_ref[0, 0] += jnp.sum(w * jnp.sum(sd * sd, axis=1, keepdims=True))

    @pl.when(i == ECH_ - 1)
    def _fin():
        r = lax.broadcasted_iota(jnp.int32, (BK_, BK_), 0)
        c = lax.broadcasted_iota(jnp.int32, (BK_, BK_), 1)
        eye = jnp.where(r == c, 1.0, 0.0).astype(jnp.float32)
        adj_raw = adj_ref[...]
        trace = jnp.sum(adj_raw * eye)
        adj = adj_raw * (1.0 - eye)
        srow = jnp.sum(adj, axis=1, keepdims=True)
        inv = 1.0 / (jnp.sqrt(srow) + 1e-12)
        inv_row = jnp.sum(eye * inv, axis=0, keepdims=True)
        adjn_ref[...] = adj * inv * inv_row
        r8 = lax.broadcasted_iota(jnp.int32, (8, 128), 0)
        c8 = lax.broadcasted_iota(jnp.int32, (8, 128), 1)
        scal_ref[...] = (jnp.where((r8 == 0) & (c8 == 0), trace, 0.0)
                         + jnp.where((r8 == 0) & (c8 == 1), den_ref[0, 0],
                                     0.0))


def _edge_kernel(gs, gd, w2):
    return pl.pallas_call(
        _edge_body,
        grid=(ECH_,),
        in_specs=[
            pl.BlockSpec((EC_BLK, DA_), lambda i: (i, 0)),
            pl.BlockSpec((EC_BLK, DA_), lambda i: (i, 0)),
            pl.BlockSpec((EC_BLK, 1), lambda i: (i, 0)),
        ],
        out_specs=[
            pl.BlockSpec((BK_, BK_), lambda i: (0, 0)),
            pl.BlockSpec((8, 128), lambda i: (0, 0)),
        ],
        out_shape=[
            jax.ShapeDtypeStruct((BK_, BK_), jnp.float32),
            jax.ShapeDtypeStruct((8, 128), jnp.float32),
        ],
        scratch_shapes=[
            pltpu.VMEM((BK_, BK_), jnp.float32),
            pltpu.SMEM((1, 1), jnp.float32),
        ],
    )(gs, gd, w2)


_SELU_ALPHA = 1.6732632423543772
_SELU_SCALE = 1.0507009873554805


def _node_body(saug_ref, x_ref, bat_ref,
               outx_ref, ortho_ref, cc_acc, xc_acc):
    i = pl.program_id(0)

    @pl.when(i == 0)
    def _init():
        cc_acc[...] = jnp.zeros((BK_, BK_), jnp.float32)
        xc_acc[...] = jnp.zeros((BK_, D_), jnp.float32)

    s = saug_ref[:, :K_]
    bat = bat_ref[...]
    colb = (lax.broadcasted_iota(jnp.int32, (1, BK_), 1) // K_
            ).astype(jnp.float32)
    cfull = jnp.where(colb == bat, jnp.concatenate([s] * B_, axis=1), 0.0)
    cc_acc[...] += lax.dot_general(
        cfull, cfull, (((0,), (0,)), ((), ())),
        preferred_element_type=jnp.float32)
    xc_acc[...] += lax.dot_general(
        cfull, x_ref[...], (((0,), (0,)), ((), ())),
        preferred_element_type=jnp.float32)

    @pl.when(i == NCH_ - 1)
    def _fin():
        r = lax.broadcasted_iota(jnp.int32, (BK_, BK_), 0)
        c = lax.broadcasted_iota(jnp.int32, (BK_, BK_), 1)
        eye = jnp.where(r == c, 1.0, 0.0).astype(jnp.float32)
        cc = cc_acc[...]
        seg = jnp.where(
            lax.broadcasted_iota(jnp.int32, (B_, BK_), 0)
            == lax.broadcasted_iota(jnp.int32, (B_, BK_), 1) // K_,
            1.0, 0.0).astype(jnp.float32)
        rowsq = jnp.sum(cc * cc, axis=1, keepdims=True)
        norm = jnp.sqrt(lax.dot_general(
            seg, rowsq, (((1,), (0,)), ((), ())),
            preferred_element_type=jnp.float32))
        col_scale = jnp.sum(seg * (1.0 / norm), axis=0,
                            keepdims=True)
        ccs = cc * col_scale
        dmat = ccs - eye / jnp.sqrt(jnp.float32(K_))
        rowsd = jnp.sum(dmat * dmat, axis=1, keepdims=True)
        segd = lax.dot_general(seg, rowsd, (((1,), (0,)), ((), ())),
                               preferred_element_type=jnp.float32)
        ortho = jnp.mean(jnp.sqrt(segd))
        r8 = lax.broadcasted_iota(jnp.int32, (8, 128), 0)
        c8 = lax.broadcasted_iota(jnp.int32, (8, 128), 1)
        ortho_ref[...] = jnp.where((r8 == 0) & (c8 == 0), ortho, 0.0)
        xc = xc_acc[...]
        outx_ref[...] = _SELU_SCALE * jnp.where(
            xc > 0.0, xc, _SELU_ALPHA * (jnp.exp(xc) - 1.0))


def _node_kernel(s_aug, x, batch_f):
    return pl.pallas_call(
        _node_body,
        grid=(NCH_,),
        in_specs=[
            pl.BlockSpec((NC_BLK, DA_), lambda i: (i, 0)),
            pl.BlockSpec((NC_BLK, D_), lambda i: (i, 0)),
            pl.BlockSpec((NC_BLK, 1), lambda i: (i, 0)),
        ],
        out_specs=[
            pl.BlockSpec((BK_, D_), lambda i: (0, 0)),
            pl.BlockSpec((8, 128), lambda i: (0, 0)),
        ],
        out_shape=[
            jax.ShapeDtypeStruct((BK_, D_), jnp.float32),
            jax.ShapeDtypeStruct((8, 128), jnp.float32),
        ],
        scratch_shapes=[
            pltpu.VMEM((BK_, BK_), jnp.float32),
            pltpu.VMEM((BK_, D_), jnp.float32),
        ],
    )(s_aug, x, batch_f)


def kernel(x, edge_index, edge_weight, batch, W, b):
    src = edge_index[0]
    dst = edge_index[1]
    batch_f = batch.astype(jnp.float32).reshape(N_, 1)
    b2 = b.reshape(1, K_)
    s_aug = _s_kernel(x, W, b2, batch_f)
    gs, gd = _gather_kernel(s_aug, src, dst)
    w2 = edge_weight.reshape(E_, 1)
    adj_norm, scal = _edge_kernel(gs, gd, w2)
    mincut_loss = -scal[0, 0] / scal[0, 1]
    out_x, ortho_scal = _node_kernel(s_aug, x, batch_f)
    return out_x, adj_norm, mincut_loss, ortho_scal[0, 0]
